# baseline (speedup 1.0000x reference)
import functools

import jax
import jax.numpy as jnp
from jax import lax
from jax.experimental import pallas as pl
from jax.experimental.pallas import tpu as pltpu
from jax.experimental.pallas import tpu_sc as plsc

_C = 2048
_Q = 4
_L = 128


def _make_sc_plane_gather(f: int, d: int, v: int, b: int):
    info = plsc.get_sparse_core_info()
    nw = info.num_subcores
    n_planes = f * d
    assert n_planes % nw == 0 and b % _C == 0
    ppw = n_planes // nw
    nc = b // _C

    mesh = plsc.VectorSubcoreMesh(core_axis_name="c", subcore_axis_name="s",
                                  num_cores=1)

    @functools.partial(
        pl.kernel,
        mesh=mesh,
        compiler_params=pltpu.CompilerParams(use_tc_tiling_on_sc=False),
        out_type=jax.ShapeDtypeStruct((n_planes, b), jnp.float32),
        scratch_types=[
            pltpu.VMEM((nc, _C), jnp.int32),
            pltpu.VMEM((b,), jnp.float32),
            pltpu.VMEM((b,), jnp.float32),
            pltpu.SemaphoreType.DMA,
            pltpu.SemaphoreType.DMA,
            pltpu.SemaphoreType.DMA,
        ],
    )
    def sc_gather(idx_hbm, tab_hbm, out_hbm, idx_v, out_a, out_b, gsem,
                  wsem_a, wsem_b):
        wid = lax.axis_index("s")
        base = wid * ppw
        bufs = (out_a, out_b)
        wsems = (wsem_a, wsem_b)
        pend = [None, None]
        for j in range(ppw):
            p = base + j
            row = tab_hbm.at[p]
            if j == 0:
                pltpu.sync_copy(idx_hbm.at[p // d], idx_v)
            else:
                @pl.when(p % d == 0)
                def _():
                    pltpu.sync_copy(idx_hbm.at[p // d], idx_v)
            buf = bufs[j % 2]
            if pend[j % 2] is not None:
                pend[j % 2].wait()

            def fire_drain(c, carry, row=row, buf=buf):
                pltpu.async_copy(
                    row.at[idx_v.at[c]], buf.at[pl.ds(c * _C, _C)], gsem)

                @pl.when(c >= _Q)
                def _():
                    pltpu.make_async_copy(
                        row.at[idx_v.at[c - _Q]],
                        buf.at[pl.ds((c - _Q) * _C, _C)], gsem).wait()
                return carry

            lax.fori_loop(0, nc, fire_drain, 0)

            def drain(c, carry, row=row, buf=buf):
                pltpu.make_async_copy(
                    row.at[idx_v.at[c]], buf.at[pl.ds(c * _C, _C)],
                    gsem).wait()
                return carry

            lax.fori_loop(nc - _Q, nc, drain, 0)
            pend[j % 2] = pltpu.async_copy(buf, out_hbm.at[p], wsems[j % 2])
        for cp in pend:
            if cp is not None:
                cp.wait()

    return sc_gather


def _mlp_t_body(xn_ref, cat_ref, gamma_ref, beta_ref, w1n_ref, w1c_ref,
                b1_ref, w2_ref, b2_ref, w3_ref, b3_ref, out_ref, *, bb: int):
    i = pl.program_id(0)
    xn = xn_ref[...]
    mean = jnp.mean(xn, axis=1, keepdims=True)
    var = jnp.mean(jnp.square(xn - mean), axis=1, keepdims=True)
    inv = lax.rsqrt(var + 1e-5)
    xb = xn_ref[:, pl.ds(i * bb, bb)]
    xb = (xb - mean) * (inv * gamma_ref[...]) + beta_ref[...]
    h = jnp.dot(w1n_ref[...], xb, preferred_element_type=jnp.float32)
    w1c = w1c_ref[...]
    hc = [jnp.dot(w1c, cat_ref[:, c, :], preferred_element_type=jnp.float32)
          for c in range(bb // _L)]
    h = h + jnp.concatenate(hc, axis=1)
    h = jnp.maximum(h + b1_ref[...], 0.0)
    h = jnp.maximum(
        jnp.dot(w2_ref[...], h, preferred_element_type=jnp.float32)
        + b2_ref[...], 0.0)
    out_ref[...] = (jnp.dot(w3_ref[...], h, preferred_element_type=jnp.float32)
                    + b3_ref[...])


def kernel(x_num, x_cat, tables, bn_gamma, bn_beta, W1, b1, W2, b2, W3, b3):
    B, NUM = x_num.shape
    F, V, D = tables.shape
    H = W2.shape[0]
    FD = F * D

    xnT = x_num.T
    idxT = x_cat.T.reshape(F, B // _C, _C)
    planes = tables.transpose(0, 2, 1).reshape(FD, V)

    catT = _make_sc_plane_gather(F, D, V, B)(idxT, planes)
    cat3 = catT.reshape(FD, B // _L, _L)

    bb = 2048
    grid = (B // bb,)
    outT = pl.pallas_call(
        functools.partial(_mlp_t_body, bb=bb),
        grid=grid,
        in_specs=[
            pl.BlockSpec((NUM, B), lambda i: (0, 0)),
            pl.BlockSpec((FD, bb // _L, _L), lambda i: (0, i, 0)),
            pl.BlockSpec((NUM, 1), lambda i: (0, 0)),
            pl.BlockSpec((NUM, 1), lambda i: (0, 0)),
            pl.BlockSpec((H, NUM), lambda i: (0, 0)),
            pl.BlockSpec((H, FD), lambda i: (0, 0)),
            pl.BlockSpec((H, 1), lambda i: (0, 0)),
            pl.BlockSpec((H, H), lambda i: (0, 0)),
            pl.BlockSpec((H, 1), lambda i: (0, 0)),
            pl.BlockSpec((1, H), lambda i: (0, 0)),
            pl.BlockSpec((1, 1), lambda i: (0, 0)),
        ],
        out_specs=pl.BlockSpec((1, bb), lambda i: (0, i)),
        out_shape=jax.ShapeDtypeStruct((1, B), jnp.float32),
        compiler_params=pltpu.CompilerParams(
            dimension_semantics=("arbitrary",)),
    )(xnT, cat3, bn_gamma.reshape(NUM, 1), bn_beta.reshape(NUM, 1),
      W1[:NUM].T, W1[NUM:].T, b1.reshape(H, 1), W2.T, b2.reshape(H, 1),
      W3.T, b3.reshape(1, 1))
    return outT.reshape(B, 1)


# speedup vs baseline: 1.4352x; 1.4352x over previous
import functools

import jax
import jax.numpy as jnp
from jax import lax
from jax.experimental import pallas as pl
from jax.experimental.pallas import tpu as pltpu
from jax.experimental.pallas import tpu_sc as plsc

_C = 4096
_Q = 4
_L = 128


def _make_sc_plane_gather(n_planes: int, d: int, v: int, b: int):
    info = plsc.get_sparse_core_info()
    nw = info.num_cores * info.num_subcores
    assert n_planes % nw == 0 and b % _C == 0
    ppw = n_planes // nw
    nc = b // _C

    mesh = plsc.VectorSubcoreMesh(core_axis_name="c", subcore_axis_name="s")

    @functools.partial(
        pl.kernel,
        mesh=mesh,
        compiler_params=pltpu.CompilerParams(use_tc_tiling_on_sc=False),
        out_type=jax.ShapeDtypeStruct((n_planes, b), jnp.float32),
        scratch_types=[
            pltpu.VMEM((nc, _C), jnp.int32),
            pltpu.VMEM((b,), jnp.float32),
            pltpu.VMEM((b,), jnp.float32),
            pltpu.SemaphoreType.DMA,
            pltpu.SemaphoreType.DMA,
            pltpu.SemaphoreType.DMA,
        ],
    )
    def sc_gather(idx_hbm, tab_hbm, out_hbm, idx_v, out_a, out_b, gsem,
                  wsem_a, wsem_b):
        wid = lax.axis_index("s") * info.num_cores + lax.axis_index("c")
        base = wid * ppw
        bufs = (out_a, out_b)
        wsems = (wsem_a, wsem_b)
        pend = [None, None]
        for j in range(ppw):
            p = base + j
            row = tab_hbm.at[p]
            if j == 0:
                pltpu.sync_copy(idx_hbm.at[p // d], idx_v)
            else:
                @pl.when(p % d == 0)
                def _():
                    pltpu.sync_copy(idx_hbm.at[p // d], idx_v)
            buf = bufs[j % 2]
            if pend[j % 2] is not None:
                pend[j % 2].wait()

            def fire_drain(c, carry, row=row, buf=buf):
                pltpu.async_copy(
                    row.at[idx_v.at[c]], buf.at[pl.ds(c * _C, _C)], gsem)

                @pl.when(c >= _Q)
                def _():
                    pltpu.make_async_copy(
                        row.at[idx_v.at[c - _Q]],
                        buf.at[pl.ds((c - _Q) * _C, _C)], gsem).wait()
                return carry

            lax.fori_loop(0, nc, fire_drain, 0)

            def drain(c, carry, row=row, buf=buf):
                pltpu.make_async_copy(
                    row.at[idx_v.at[c]], buf.at[pl.ds(c * _C, _C)],
                    gsem).wait()
                return carry

            lax.fori_loop(nc - _Q, nc, drain, 0)
            pend[j % 2] = pltpu.async_copy(buf, out_hbm.at[p], wsems[j % 2])
        for cp in pend:
            if cp is not None:
                cp.wait()

    return sc_gather


def _mlp_t_body(xn_ref, cata_ref, catb_ref, gamma_ref, beta_ref, w1n_ref,
                w1ca_ref, w1cb_ref, b1_ref, w2_ref, b2_ref, w3_ref, b3_ref,
                out_ref, *, bb: int):
    i = pl.program_id(0)
    xn = xn_ref[...]
    mean = jnp.mean(xn, axis=1, keepdims=True)
    var = jnp.mean(jnp.square(xn - mean), axis=1, keepdims=True)
    inv = lax.rsqrt(var + 1e-5)
    xb = xn_ref[:, pl.ds(i * bb, bb)]
    xb = (xb - mean) * (inv * gamma_ref[...]) + beta_ref[...]
    h = jnp.dot(w1n_ref[...], xb, preferred_element_type=jnp.float32)
    for w_ref, c_ref in ((w1ca_ref, cata_ref), (w1cb_ref, catb_ref)):
        w = w_ref[...]
        hc = [jnp.dot(w, c_ref[:, c, :], preferred_element_type=jnp.float32)
              for c in range(bb // _L)]
        h = h + jnp.concatenate(hc, axis=1)
    h = jnp.maximum(h + b1_ref[...], 0.0)
    h = jnp.maximum(
        jnp.dot(w2_ref[...], h, preferred_element_type=jnp.float32)
        + b2_ref[...], 0.0)
    out_ref[...] = (jnp.dot(w3_ref[...], h, preferred_element_type=jnp.float32)
                    + b3_ref[...])


def kernel(x_num, x_cat, tables, bn_gamma, bn_beta, W1, b1, W2, b2, W3, b3):
    B, NUM = x_num.shape
    F, V, D = tables.shape
    H = W2.shape[0]
    FD = F * D

    xnT = x_num.T
    FH = F // 2
    HPL = FH * D
    idxT_a = x_cat.T[:FH].reshape(FH, B // _C, _C)
    idxT_b = x_cat.T[FH:].reshape(F - FH, B // _C, _C)
    planes_a = tables[:FH].transpose(0, 2, 1).reshape(HPL, V)
    planes_b = tables[FH:].transpose(0, 2, 1).reshape(FD - HPL, V)

    gat = _make_sc_plane_gather(HPL, D, V, B)
    catT_a = gat(idxT_a, planes_a)
    catT_b = gat(idxT_b, planes_b)
    cat3_a = catT_a.reshape(HPL, B // _L, _L)
    cat3_b = catT_b.reshape(FD - HPL, B // _L, _L)

    bb = 2048
    grid = (B // bb,)
    outT = pl.pallas_call(
        functools.partial(_mlp_t_body, bb=bb),
        grid=grid,
        in_specs=[
            pl.BlockSpec((NUM, B), lambda i: (0, 0)),
            pl.BlockSpec((HPL, bb // _L, _L), lambda i: (0, i, 0)),
            pl.BlockSpec((FD - HPL, bb // _L, _L), lambda i: (0, i, 0)),
            pl.BlockSpec((NUM, 1), lambda i: (0, 0)),
            pl.BlockSpec((NUM, 1), lambda i: (0, 0)),
            pl.BlockSpec((H, NUM), lambda i: (0, 0)),
            pl.BlockSpec((H, HPL), lambda i: (0, 0)),
            pl.BlockSpec((H, FD - HPL), lambda i: (0, 0)),
            pl.BlockSpec((H, 1), lambda i: (0, 0)),
            pl.BlockSpec((H, H), lambda i: (0, 0)),
            pl.BlockSpec((H, 1), lambda i: (0, 0)),
            pl.BlockSpec((1, H), lambda i: (0, 0)),
            pl.BlockSpec((1, 1), lambda i: (0, 0)),
        ],
        out_specs=pl.BlockSpec((1, bb), lambda i: (0, i)),
        out_shape=jax.ShapeDtypeStruct((1, B), jnp.float32),
        compiler_params=pltpu.CompilerParams(
            dimension_semantics=("arbitrary",)),
    )(xnT, cat3_a, cat3_b, bn_gamma.reshape(NUM, 1), bn_beta.reshape(NUM, 1),
      W1[:NUM].T, W1[NUM:NUM + HPL].T, W1[NUM + HPL:].T, b1.reshape(H, 1),
      W2.T, b2.reshape(H, 1), W3.T, b3.reshape(1, 1))
    return outT.reshape(B, 1)
